# baseline (speedup 1.0000x reference)
import jax
import jax.numpy as jnp
from jax import lax
from jax.experimental import pallas as pl
from jax.experimental.pallas import tpu as pltpu
from jax.experimental.pallas import tpu_sc as plsc

NC = 2
NS = 16
NW = NC * NS
B = 16384
D = 64
NROW = 1000000
SCOLS = 512
NT_TOTAL = (NROW + SCOLS - 1) // SCOLS
TPW = (NT_TOTAL + NW - 1) // NW
SH = 9
SROWS = B + 16
DUMMY = B + 8
DUMMY_R = 0x40000000
CHUNK = 128
CVREG = CHUNK // 16
SPARE = CHUNK - 1
CAP = CHUNK - 16


def _iota16():
    return lax.iota(jnp.int32, 16)


def _pick(idx):
    return jnp.clip(idx, 0, 15)


def _merge16(pend, comp, pcnt):
    i = _iota16()
    shifted = comp.at[_pick(i - pcnt)].get(mode="promise_in_bounds")
    merged = jnp.where(i < pcnt, pend, shifted)
    leftover = comp.at[_pick(i + 16 - pcnt)].get(mode="promise_in_bounds")
    return merged, leftover


def _extract_side(table_hbm, tail_hbm, idx_hbm, out_hbm, refs):
    (all_v, own_e, e_ord, slab, rowbuf, uloc, tmp16,
     sem_a, sem_b) = refs

    wid = lax.axis_index("s") * NC + lax.axis_index("c")
    lo_t = wid * TPW
    hi_t = jnp.minimum(lo_t + TPW, NT_TOTAL)
    nt = hi_t - lo_t

    pltpu.sync_copy(idx_hbm, all_v)

    def compress16(vals, mask):
        plsc.store_compressed(tmp16.at[pl.ds(0, 16)], vals, mask=mask)
        return tmp16[...]

    def compact_step(v, carry):
        pend_r, pend_e, pcnt, wcnt = carry
        x = all_v[pl.ds(v * 16, 16)]
        t = lax.shift_right_logical(x, SH)
        m = (t >= lo_t) & (t < hi_t)
        nh = plsc.all_reduce_population_count(m)[0]

        def with_hits(carry):
            pend_r, pend_e, pcnt, wcnt = carry
            comp_r = compress16(x, m)
            e = v * 16 + _iota16()
            comp_e = compress16(e, m)
            mer_r, left_r = _merge16(pend_r, comp_r, pcnt)
            mer_e, left_e = _merge16(pend_e, comp_e, pcnt)
            total = pcnt + nh

            def flush(args):
                mer_r, mer_e, left_r, left_e, wcnt = args
                all_v[pl.ds(wcnt * 16, 16)] = mer_r
                own_e[pl.ds(wcnt * 16, 16)] = mer_e
                return left_r, left_e, wcnt + 1

            pend_r, pend_e, wcnt = lax.cond(
                total >= 16, flush,
                lambda args: (args[0], args[1], args[4]),
                (mer_r, mer_e, left_r, left_e, wcnt))
            pcnt = jnp.where(total >= 16, total - 16, total)
            return pend_r, pend_e, pcnt, wcnt

        return lax.cond(nh > 0, with_hits, lambda c: c,
                        (pend_r, pend_e, pcnt, wcnt))

    zero16 = jnp.zeros((16,), jnp.int32)
    pend_r, pend_e, pcnt, wcnt = lax.fori_loop(
        0, B // 16, compact_step,
        (zero16, zero16, jnp.int32(0), jnp.int32(0)))

    @pl.when(pcnt > 0)
    def _():
        all_v[pl.ds(wcnt * 16, 16)] = jnp.where(
            _iota16() < pcnt, pend_r, DUMMY_R)
        own_e[pl.ds(wcnt * 16, 16)] = jnp.where(
            _iota16() < pcnt, pend_e, DUMMY)

    cnt = wcnt * 16 + pcnt
    nv = lax.div(cnt + 15, 16)

    def fetch(jt_local, buf, sem):
        jt = lo_t + jt_local

        @pl.when(jt < NT_TOTAL - 1)
        def _():
            col = pl.multiple_of(jt * SCOLS, 128)
            pltpu.async_copy(
                table_hbm.at[:, pl.ds(col, SCOLS)], slab.at[buf], sem)

        @pl.when(jt == NT_TOTAL - 1)
        def _():
            pltpu.async_copy(tail_hbm, slab.at[buf], sem)

    def wait_fetch(buf, sem):
        pltpu.make_async_copy(
            table_hbm.at[:, pl.ds(0, SCOLS)], slab.at[buf], sem).wait()

    dummy_vreg = jnp.full((16,), DUMMY, jnp.int32)

    def flush_group(ewl, gc, ge):
        slot_v = lax.shift_right_logical(gc, SH)
        c_v = gc & (SCOLS - 1)
        for d0 in range(0, D, 8):
            grabbed = [
                plsc.load_gather(
                    slab, [slot_v, jnp.full((16,), d, jnp.int32), c_v])
                for d in range(d0, d0 + 8)]
            for j, d in enumerate(range(d0, d0 + 8)):
                plsc.store_scatter(
                    rowbuf, [_iota16(), jnp.full((16,), d, jnp.int32)],
                    grabbed[j])
        base = ewl * 16
        for i in range(16):
            for k in range(4):
                uloc[base + i, pl.ds(k * 16, 16)] = (
                    rowbuf[i, pl.ds(k * 16, 16)])
        e_ord[0, pl.ds(ewl * 16, 16)] = ge
        ewl = ewl + 1

        def do_scatter(ewl):
            pltpu.sync_copy(uloc, out_hbm.at[e_ord.at[0]])
            return jnp.int32(0)

        return lax.cond(ewl >= CVREG, do_scatter, lambda e: e, ewl)

    def flush_pending(carry):
        ewl, pcx, pend_c, pend_e = carry

        def go(args):
            ewl, pcx, pend_c, pend_e = args
            gc = jnp.where(_iota16() < pcx, pend_c, 0)
            ge = jnp.where(_iota16() < pcx, pend_e, DUMMY)
            ewl = flush_group(ewl, gc, ge)
            return ewl, jnp.int32(0), pend_c, pend_e

        return lax.cond(pcx > 0, go, lambda a: a,
                        (ewl, pcx, pend_c, pend_e))

    def scan_tile(jt_local, buf, carry):
        jt = lo_t + jt_local

        def vstep(v, carry):
            rv = all_v[pl.ds(v * 16, 16)]
            hit = lax.shift_right_logical(rv, SH) == jt

            def process(carry):
                ewl, pcx, pend_c, pend_e = carry
                ev = own_e[pl.ds(v * 16, 16)]
                nh = plsc.all_reduce_population_count(hit)[0]
                cc = (rv & (SCOLS - 1)) + buf * SCOLS
                comp_c = compress16(cc, hit)
                comp_e = compress16(ev, hit)
                mer_c, left_c = _merge16(pend_c, comp_c, pcx)
                mer_e, left_e = _merge16(pend_e, comp_e, pcx)
                total = pcx + nh

                def full(args):
                    ewl, mer_c, mer_e, left_c, left_e = args
                    ewl = flush_group(ewl, mer_c, mer_e)
                    return ewl, left_c, left_e

                ewl, pend_c, pend_e = lax.cond(
                    total >= 16, full, lambda a: (a[0], a[1], a[2]),
                    (ewl, mer_c, mer_e, left_c, left_e))
                pcx = jnp.where(total >= 16, total - 16, total)
                return ewl, pcx, pend_c, pend_e

            return lax.cond(jnp.any(hit), process, lambda c_: c_, carry)

        return lax.fori_loop(0, nv, vstep, carry)

    fetch(jnp.int32(0), 0, sem_a)
    npair = lax.div(nt + 1, 2)

    def pair_step(p, carry):
        j0 = 2 * p
        j1 = 2 * p + 1

        def prefetch1(car):
            car = flush_pending(car)
            fetch(j1, 1, sem_b)
            return car

        carry = lax.cond(j1 < nt, prefetch1, lambda c: c, carry)
        wait_fetch(0, sem_a)
        carry = scan_tile(j0, 0, carry)

        def prefetch0(car):
            car = flush_pending(car)
            fetch(j0 + 2, 0, sem_a)
            return car

        carry = lax.cond(j0 + 2 < nt, prefetch0, lambda c: c, carry)

        def do_second(car):
            wait_fetch(1, sem_b)
            return scan_tile(j1, 1, car)

        return lax.cond(j1 < nt, do_second, lambda car: car, carry)

    carry = lax.fori_loop(
        0, npair, pair_step,
        (jnp.int32(0), jnp.int32(0), zero16, zero16))
    ewl, _, _, _ = flush_pending(carry)

    @pl.when(ewl > 0)
    def _():
        def pad(j, _):
            e_ord[0, pl.ds((ewl + j) * 16, 16)] = dummy_vreg
            return _

        lax.fori_loop(0, CVREG - ewl, pad, 0)
        pltpu.sync_copy(uloc, out_hbm.at[e_ord.at[0]])


def _body_a(xu, xm, tu, tm, iu, im, U, M,
            all_v, own_e, e_ord, slab, rowbuf, uloc, tmp16,
            sem_a, sem_b):
    refs = (all_v, own_e, e_ord, slab, rowbuf, uloc, tmp16,
            sem_a, sem_b)
    _extract_side(xu, tu, iu, U, refs)
    _extract_side(xm, tm, im, M, refs)


@jax.jit
def _run_a(xut, xmt, tail_u, tail_m, iu, im):
    mesh = plsc.VectorSubcoreMesh(
        core_axis_name="c", subcore_axis_name="s",
        num_cores=NC, num_subcores=NS)
    f = pl.kernel(
        _body_a,
        out_type=(jax.ShapeDtypeStruct((SROWS, 128), jnp.float32),
                  jax.ShapeDtypeStruct((SROWS, 128), jnp.float32)),
        mesh=mesh,
        scratch_types=[
            pltpu.VMEM((B,), jnp.int32),
            pltpu.VMEM((B,), jnp.int32),
            pltpu.VMEM((1, CHUNK), jnp.int32),
            pltpu.VMEM((2, D, SCOLS), jnp.float32),
            pltpu.VMEM((16, 136), jnp.float32),
            pltpu.VMEM((CHUNK, 128), jnp.float32),
            pltpu.VMEM((16,), jnp.int32),
            pltpu.SemaphoreType.DMA,
            pltpu.SemaphoreType.DMA,
        ],
        compiler_params=pltpu.CompilerParams(
            needs_layout_passes=False, use_tc_tiling_on_sc=True),
    )
    return f(xut, xmt, tail_u, tail_m, iu, im)


def _body_b(U, M, out_hbm, ub, mb, ob, sem):
    wid = lax.axis_index("s") * NC + lax.axis_index("c")
    base = wid * (B // NW)

    def chunk_step(ci, _):
        row0 = base + ci * 128
        cp_u = pltpu.async_copy(U.at[pl.ds(row0, 128), :], ub, sem)
        cp_m = pltpu.async_copy(M.at[pl.ds(row0, 128), :], mb, sem)
        cp_u.wait()
        cp_m.wait()

        def grp(g, _):
            res = jnp.zeros((16,), jnp.float32)
            for i in range(16):
                pos = g * 16 + i
                s = jnp.zeros((16,), jnp.float32)
                for k in range(4):
                    s = s + (ub[pos, pl.ds(k * 16, 16)] *
                             mb[pos, pl.ds(k * 16, 16)])
                tot = jnp.sum(s)
                res = jnp.where(_iota16() == i, tot, res)
            ob[pl.ds(ci * 128 + g * 16, 16)] = res
            return _

        lax.fori_loop(0, 8, grp, 0)
        return _

    lax.fori_loop(0, 4, chunk_step, 0)
    pltpu.sync_copy(ob, out_hbm.at[pl.ds(base, B // NW)])


@jax.jit
def _run_b(U, M):
    mesh = plsc.VectorSubcoreMesh(
        core_axis_name="c", subcore_axis_name="s",
        num_cores=NC, num_subcores=NS)
    f = pl.kernel(
        _body_b,
        out_type=jax.ShapeDtypeStruct((B,), jnp.float32),
        mesh=mesh,
        scratch_types=[
            pltpu.VMEM((128, 128), jnp.float32),
            pltpu.VMEM((128, 128), jnp.float32),
            pltpu.VMEM((B // NW,), jnp.float32),
            pltpu.SemaphoreType.DMA,
        ],
        compiler_params=pltpu.CompilerParams(
            needs_layout_passes=False, use_tc_tiling_on_sc=True),
    )
    return f(U, M)


def kernel(x_user, x_movie, edge_label_index):
    idx = edge_label_index.astype(jnp.int32)
    xut = x_user.T
    xmt = x_movie.T
    ntail = NROW - (NT_TOTAL - 1) * SCOLS
    tail_u = jnp.pad(xut[:, (NT_TOTAL - 1) * SCOLS:],
                     ((0, 0), (0, SCOLS - ntail)))
    tail_m = jnp.pad(xmt[:, (NT_TOTAL - 1) * SCOLS:],
                     ((0, 0), (0, SCOLS - ntail)))
    U, M = _run_a(xut, xmt, tail_u, tail_m, idx[0], idx[1])
    return _run_b(U, M)


# speedup vs baseline: 1.6057x; 1.6057x over previous
import jax
import jax.numpy as jnp
from jax import lax
from jax.experimental import pallas as pl
from jax.experimental.pallas import tpu as pltpu
from jax.experimental.pallas import tpu_sc as plsc

NC = 2
NS = 16
NW = NC * NS
B = 16384
D = 64
BPW = B // NW
CHUNK = 128
NCHUNK = BPW // CHUNK
HALF = BPW // 2
HCHUNK = HALF // CHUNK


def _iota16():
    return lax.iota(jnp.int32, 16)


def _sc_body(xu_hbm, xm_hbm, iu_hbm, im_hbm, pu_hbm, pm_hbm, out_hbm,
             iu_v, im_v, pu_v, pm_v, ur_v, mr_v, o_v, sem_u, sem_m):
    wid = lax.axis_index("s") * NC + lax.axis_index("c")
    base = wid * BPW

    pltpu.sync_copy(iu_hbm.at[wid], iu_v)
    pltpu.sync_copy(im_hbm.at[wid], im_v)
    pltpu.sync_copy(pu_hbm.at[wid], pu_v)
    pltpu.sync_copy(pm_hbm.at[wid], pm_v)

    for h in range(2):
        copies = []
        for j in range(HCHUNK):
            jc = h * HCHUNK + j
            copies.append(pltpu.async_copy(
                xu_hbm.at[iu_v.at[jc]],
                ur_v.at[pl.ds(j * CHUNK, CHUNK)], sem_u))
            copies.append(pltpu.async_copy(
                xm_hbm.at[im_v.at[jc]],
                mr_v.at[pl.ds(j * CHUNK, CHUNK)], sem_m))
        for cp in copies:
            cp.wait()

        def group_step(g, carry):
            e_ids = g * 16 + _iota16()
            pos = h * HALF + g * 16
            pu = pu_v[lax.div(pos, CHUNK), pl.ds(lax.rem(pos, CHUNK), 16)]
            pm = pm_v[lax.div(pos, CHUNK), pl.ds(lax.rem(pos, CHUNK), 16)]
            cu0 = pu * D
            cm0 = pm * D

            def dim_step(d, acc):
                gu = plsc.load_gather(ur_v, [e_ids, cu0 + d])
                gm = plsc.load_gather(mr_v, [e_ids, cm0 + d])
                return acc + gu * gm

            acc = lax.fori_loop(0, D, dim_step, jnp.zeros((16,), jnp.float32))
            o_v[pl.ds(h * HALF + g * 16, 16)] = acc
            return carry

        lax.fori_loop(0, HALF // 16, group_step, 0)

    pltpu.sync_copy(o_v, out_hbm.at[pl.ds(base, BPW)])


@jax.jit
def _run(xp_u, xp_m, iu, im, pu, pm):
    mesh = plsc.VectorSubcoreMesh(
        core_axis_name="c", subcore_axis_name="s",
        num_cores=NC, num_subcores=NS)
    f = pl.kernel(
        _sc_body,
        out_type=jax.ShapeDtypeStruct((B,), jnp.float32),
        mesh=mesh,
        scratch_types=[
            pltpu.VMEM((NCHUNK, CHUNK), jnp.int32),
            pltpu.VMEM((NCHUNK, CHUNK), jnp.int32),
            pltpu.VMEM((NCHUNK, CHUNK), jnp.int32),
            pltpu.VMEM((NCHUNK, CHUNK), jnp.int32),
            pltpu.VMEM((HALF, 128), jnp.float32),
            pltpu.VMEM((HALF, 128), jnp.float32),
            pltpu.VMEM((BPW,), jnp.float32),
            pltpu.SemaphoreType.DMA,
            pltpu.SemaphoreType.DMA,
        ],
        compiler_params=pltpu.CompilerParams(
            needs_layout_passes=False, use_tc_tiling_on_sc=True),
    )
    return f(xp_u, xp_m, iu, im, pu, pm)


def kernel(x_user, x_movie, edge_label_index):
    idx = edge_label_index.astype(jnp.int32)
    xp_u = x_user.reshape(500000, 128)
    xp_m = x_movie.reshape(500000, 128)
    iu = (idx[0] >> 1).reshape(NW, NCHUNK, CHUNK)
    im = (idx[1] >> 1).reshape(NW, NCHUNK, CHUNK)
    pu = (idx[0] & 1).reshape(NW, NCHUNK, CHUNK)
    pm = (idx[1] & 1).reshape(NW, NCHUNK, CHUNK)
    return _run(xp_u, xp_m, iu, im, pu, pm)


# speedup vs baseline: 1.6291x; 1.0146x over previous
import jax
import jax.numpy as jnp
from jax import lax
from jax.experimental import pallas as pl
from jax.experimental.pallas import tpu as pltpu
from jax.experimental.pallas import tpu_sc as plsc

NC = 2
NS = 16
NW = NC * NS
B = 16384
D = 64
BPW = B // NW
CHUNK = 128
NCHUNK = BPW // CHUNK


def _sc_body(xu_hbm, xm_hbm, iu_hbm, im_hbm, out_hbm,
             iu_v, im_v, ur_v, mr_v, o_v, sem_u, sem_m):
    wid = lax.axis_index("s") * NC + lax.axis_index("c")
    base = wid * BPW

    pltpu.sync_copy(iu_hbm.at[wid], iu_v)
    pltpu.sync_copy(im_hbm.at[wid], im_v)

    copies = []
    for j in range(NCHUNK):
        copies.append(pltpu.async_copy(
            xu_hbm.at[iu_v.at[j]], ur_v.at[pl.ds(j * CHUNK, CHUNK)], sem_u))
        copies.append(pltpu.async_copy(
            xm_hbm.at[im_v.at[j]], mr_v.at[pl.ds(j * CHUNK, CHUNK)], sem_m))
    for cp in copies:
        cp.wait()

    def group_step(g, carry):
        e_ids = g * 16 + lax.iota(jnp.int32, 16)

        def dim_step(d, acc):
            d_ids = jnp.full((16,), d, jnp.int32)
            pu = plsc.load_gather(ur_v, [e_ids, d_ids])
            pm = plsc.load_gather(mr_v, [e_ids, d_ids])
            return acc + pu * pm

        acc = lax.fori_loop(0, D, dim_step, jnp.zeros((16,), jnp.float32))
        o_v[pl.ds(g * 16, 16)] = acc
        return carry

    lax.fori_loop(0, BPW // 16, group_step, 0)

    pltpu.sync_copy(o_v, out_hbm.at[pl.ds(base, BPW)])


@jax.jit
def _run(x_user, x_movie, iu, im):
    mesh = plsc.VectorSubcoreMesh(
        core_axis_name="c", subcore_axis_name="s",
        num_cores=NC, num_subcores=NS)
    f = pl.kernel(
        _sc_body,
        out_type=jax.ShapeDtypeStruct((B,), jnp.float32),
        mesh=mesh,
        scratch_types=[
            pltpu.VMEM((NCHUNK, CHUNK), jnp.int32),
            pltpu.VMEM((NCHUNK, CHUNK), jnp.int32),
            pltpu.VMEM((BPW, D), jnp.float32),
            pltpu.VMEM((BPW, D), jnp.float32),
            pltpu.VMEM((BPW,), jnp.float32),
            pltpu.SemaphoreType.DMA,
            pltpu.SemaphoreType.DMA,
        ],
        compiler_params=pltpu.CompilerParams(
            needs_layout_passes=False, use_tc_tiling_on_sc=False),
    )
    return f(x_user, x_movie, iu, im)


def kernel(x_user, x_movie, edge_label_index):
    idx = edge_label_index.astype(jnp.int32)
    iu = idx[0].reshape(NW, NCHUNK, CHUNK)
    im = idx[1].reshape(NW, NCHUNK, CHUNK)
    return _run(x_user, x_movie, iu, im)
